# Initial kernel scaffold; baseline (speedup 1.0000x reference)
import jax
import jax.numpy as jnp
from jax.experimental import pallas as pl


def kernel(x, pe_weight):
    raise NotImplementedError("write your pallas kernel here")


# speedup vs baseline: 1.7216x; 1.7216x over previous
import jax
import jax.numpy as jnp
import numpy as np
from jax.experimental import pallas as pl
from jax.experimental.pallas import tpu as pltpu

D_MODEL_K = 1024
SCALE_K = float(np.sqrt(D_MODEL_K))
SEQ_BLK = 256


def _body(x_ref, pe_ref, o_ref):
    o_ref[...] = x_ref[...] + pe_ref[...][None] * SCALE_K


def kernel(x, pe_weight):
    B, S, D = x.shape
    grid = (S // SEQ_BLK,)
    return pl.pallas_call(
        _body,
        grid=grid,
        in_specs=[
            pl.BlockSpec((B, SEQ_BLK, D), lambda i: (0, i, 0)),
            pl.BlockSpec((SEQ_BLK, D), lambda i: (i, 0)),
        ],
        out_specs=pl.BlockSpec((B, SEQ_BLK, D), lambda i: (0, i, 0)),
        out_shape=jax.ShapeDtypeStruct((B, S, D), x.dtype),
    )(x, pe_weight[:S])
